# baseline (speedup 1.0000x reference)
import functools
import math

import jax
import jax.numpy as jnp
from jax import lax
from jax.experimental import pallas as pl
from jax.experimental.pallas import tpu as pltpu
from jax.experimental.pallas import tpu_sc as plsc

N = 10000
E = 320000
D = 128

NC = 2
NS = 16
NW = NC * NS
EPW = E // NW
H = 2
EPWH = EPW // H
GC = 125
NGC = EPWH // GC
SCK = 40
NSC = EPWH // SCK
JB = 5
EB = EPWH // JB
EH = E // H
NB = 2000
NNB = N // NB

_mesh = plsc.VectorSubcoreMesh(core_axis_name="c", subcore_axis_name="s")
_f32 = jnp.float32
_sc_params = pltpu.CompilerParams(needs_layout_passes=False)


_NBUF = 4


@functools.partial(
    pl.kernel,
    out_type=(
        jax.ShapeDtypeStruct((NW, NGC, GC, D), _f32),
        jax.ShapeDtypeStruct((NW, NGC, GC, D), _f32),
    ),
    mesh=_mesh,
    scratch_types=[
        pltpu.VMEM((NGC, GC), jnp.int32),
        [pltpu.VMEM((GC, D), _f32)] * _NBUF,
        [pltpu.SemaphoreType.DMA] * _NBUF,
        [pltpu.SemaphoreType.DMA] * _NBUF,
    ],
)
def _gather_rows(node_hbm, src3_hbm, dst3_hbm, sr_hbm, dr_hbm,
                 idx_v, bufs, gsems, wsems):
    cid = lax.axis_index("c")
    sid = lax.axis_index("s")
    wid = sid * NC + cid

    def run(idx3_hbm, out_hbm):
        pltpu.sync_copy(idx3_hbm.at[wid], idx_v)

        def gath(j, b):
            return pltpu.make_async_copy(
                node_hbm.at[idx_v.at[j]], bufs[b], gsems[b])

        def wrb(j, b):
            return pltpu.make_async_copy(
                bufs[b], out_hbm.at[wid, j], wsems[b])

        for b in range(_NBUF):
            gath(b, b).start()

        def body(t, carry):
            j0 = _NBUF * t
            for b in range(_NBUF):
                gath(j0 + b, b).wait()
                wrb(j0 + b, b).start()
            for b in range(_NBUF):
                wrb(j0 + b, b).wait()

                @pl.when(j0 + b + _NBUF < NGC)
                def _():
                    gath(j0 + b + _NBUF, b).start()

            return carry

        lax.fori_loop(0, NGC // _NBUF, body, 0)

    run(src3_hbm, sr_hbm)
    run(dst3_hbm, dr_hbm)


def _edge_body(sr, dr, er, ws3, wd2, w1e, b1, w2t, b2, *rest):
    e_ref, ev_ref, t_ref = rest[-3:]
    s = sr[0, 0]
    d = dr[0, 0]
    ed = er[0, 0, 0]
    s3 = jnp.dot(s, ws3[...], preferred_element_type=_f32)
    d2 = jnp.dot(d, wd2[...], preferred_element_type=_f32)
    k = s3[:, :D] + ed
    v = s3[:, D:2 * D] + ed
    q = d2[:, :D]
    ones_s = jnp.full((D, D), 1.0 / math.sqrt(D), _f32)
    e2d = jnp.exp(jnp.dot(q * k, ones_s, preferred_element_type=_f32))
    e_ref[0, 0, 0, :] = e2d[:, 0]
    ev_ref[0, 0] = v * e2d
    h = s3[:, 2 * D:] + jnp.dot(ed, w1e[...], preferred_element_type=_f32)
    h = jnp.maximum(h + d2[:, D:] + b1[...], 0.0)
    t_ref[0, 0, 0] = jnp.dot(h, w2t[...], preferred_element_type=_f32) + b2[...]


def _edge_pass(h, srb, drb, er5, ws3, wd2, w1e, b1, w2t, b2, trip_in):
    eb4 = pl.BlockSpec((1, 1, EB, D), lambda w, j: (w, j, 0, 0))
    eb5 = pl.BlockSpec((1, 1, 1, EB, D), lambda w, j: (w, h, j, 0, 0))
    b_spec = pl.BlockSpec((1, D), lambda w, j: (0, 0))
    in_specs = [eb4, eb4, eb5,
                pl.BlockSpec((D, 3 * D), lambda w, j: (0, 0)),
                pl.BlockSpec((D, 2 * D), lambda w, j: (0, 0)),
                pl.BlockSpec((D, D), lambda w, j: (0, 0)),
                b_spec,
                pl.BlockSpec((D, D), lambda w, j: (0, 0)),
                b_spec]
    inputs = [srb, drb, er5, ws3, wd2, w1e, b1, w2t, b2]
    kwargs = {}
    if trip_in is not None:
        in_specs.append(pl.BlockSpec(memory_space=pl.ANY))
        inputs.append(trip_in)
        kwargs["input_output_aliases"] = {9: 2}
    return pl.pallas_call(
        _edge_body,
        grid=(NW, JB),
        in_specs=in_specs,
        out_specs=[
            pl.BlockSpec((1, 1, 1, EB), lambda w, j: (w, j, 0, 0)),
            eb4,
            eb5,
        ],
        out_shape=[
            jax.ShapeDtypeStruct((NW, JB, 1, EB), _f32),
            jax.ShapeDtypeStruct((NW, JB, EB, D), _f32),
            jax.ShapeDtypeStruct((NW, H, JB, EB, D), _f32),
        ],
        **kwargs,
    )(*inputs)


@functools.partial(
    pl.kernel,
    out_type=jax.ShapeDtypeStruct((NC, N), _f32),
    mesh=_mesh,
    scratch_types=[
        pltpu.VMEM((NSC, SCK), _f32),
        pltpu.VMEM((NSC, SCK), jnp.int32),
        pltpu.VMEM_SHARED((N,), _f32),
    ],
    compiler_params=_sc_params,
)
def _denom(e3_hbm, d3_hbm, z1_hbm, dpart_hbm, ebuf, dbuf, den_sh):
    cid = lax.axis_index("c")
    sid = lax.axis_index("s")
    wid = sid * NC + cid

    @pl.when(sid == 0)
    def _():
        pltpu.sync_copy(z1_hbm, den_sh)

    plsc.subcore_barrier()
    pltpu.sync_copy(e3_hbm.at[wid], ebuf)
    pltpu.sync_copy(d3_hbm.at[wid], dbuf)

    def body(j, carry):
        pltpu.sync_copy(ebuf.at[j], den_sh.at[dbuf.at[j]], add=True)
        return carry

    lax.fori_loop(0, NSC, body, 0)
    plsc.subcore_barrier()

    @pl.when(sid == 0)
    def _():
        pltpu.sync_copy(den_sh, dpart_hbm.at[cid])


@functools.partial(
    pl.kernel,
    out_type=jax.ShapeDtypeStruct((NC, N, D), _f32),
    mesh=_mesh,
    scratch_types=[
        pltpu.VMEM((NSC, SCK), jnp.int32),
        pltpu.VMEM((SCK, D), _f32),
        pltpu.VMEM((SCK, D), _f32),
        pltpu.VMEM_SHARED((N, D), _f32),
        pltpu.SemaphoreType.DMA,
        pltpu.SemaphoreType.DMA,
    ],
    compiler_params=_sc_params,
)
def _agg_scatter(d3_hbm, ev_hbm, zn_hbm, agg_hbm,
                 dbuf, buf_a, buf_b, agg_sh, sem_a, sem_b):
    cid = lax.axis_index("c")
    sid = lax.axis_index("s")
    wid = sid * NC + cid

    @pl.when(sid == 0)
    def _():
        pltpu.sync_copy(zn_hbm, agg_sh)

    pltpu.sync_copy(d3_hbm.at[wid], dbuf)
    plsc.subcore_barrier()

    def load(j, buf, sem):
        return pltpu.make_async_copy(
            ev_hbm.at[pl.ds(wid * EPWH + j * SCK, SCK)], buf, sem)

    load(0, buf_a, sem_a).start()
    load(1, buf_b, sem_b).start()

    def body(t, carry):
        j0 = 2 * t
        load(j0, buf_a, sem_a).wait()
        pltpu.sync_copy(buf_a, agg_sh.at[dbuf.at[j0]], add=True)

        @pl.when(j0 + 2 < NSC)
        def _():
            load(j0 + 2, buf_a, sem_a).start()

        load(j0 + 1, buf_b, sem_b).wait()
        pltpu.sync_copy(buf_b, agg_sh.at[dbuf.at[j0 + 1]], add=True)

        @pl.when(j0 + 3 < NSC)
        def _():
            load(j0 + 3, buf_b, sem_b).start()

        return carry

    lax.fori_loop(0, NSC // 2, body, 0)
    load(NSC - 1, buf_a, sem_a).wait()
    pltpu.sync_copy(buf_a, agg_sh.at[dbuf.at[NSC - 1]], add=True)

    plsc.subcore_barrier()

    @pl.when(sid == 0)
    def _():
        pltpu.sync_copy(agg_sh, agg_hbm.at[cid])


def _final_body(node, aggpa, aggpb, dpa, dpb, wot, lns, lnb, out):
    den = (dpa[0, 0, 0, :] + dpa[1, 0, 0, :]
           + dpb[0, 0, 0, :] + dpb[1, 0, 0, :])
    rden = 1.0 / jnp.maximum(den, 1e-30)
    agg = (aggpa[0] + aggpa[1] + aggpb[0] + aggpb[1]) * rden[:, None]
    pre = node[...] + jnp.dot(agg, wot[...], preferred_element_type=_f32)
    mu = jnp.mean(pre, axis=1, keepdims=True)
    ctr = pre - mu
    var = jnp.mean(ctr * ctr, axis=1, keepdims=True)
    out[...] = ctr * lax.rsqrt(var + 1e-5) * lns[...] + lnb[...]


def _final_pass(node_reps, aggpa, aggpb, dpa, dpb, wot, lns, lnb):
    agg_spec = pl.BlockSpec((NC, NB, D), lambda i: (0, i, 0))
    dp_spec = pl.BlockSpec((NC, 1, 1, NB), lambda i: (0, i, 0, 0))
    return pl.pallas_call(
        _final_body,
        grid=(NNB,),
        in_specs=[
            pl.BlockSpec((NB, D), lambda i: (i, 0)),
            agg_spec, agg_spec, dp_spec, dp_spec,
            pl.BlockSpec((D, D), lambda i: (0, 0)),
            pl.BlockSpec((1, D), lambda i: (0, 0)),
            pl.BlockSpec((1, D), lambda i: (0, 0)),
        ],
        out_specs=pl.BlockSpec((NB, D), lambda i: (i, 0)),
        out_shape=jax.ShapeDtypeStruct((N, D), _f32),
    )(node_reps, aggpa, aggpb, dpa, dpb, wot, lns, lnb)


def kernel(node_reps, edge_reps, adjacency_list, Wq, Wk, Wv, Wo,
           ln_scale, ln_bias, W1, b1, W2, b2):
    src = adjacency_list[0]
    dst = adjacency_list[1]
    src4 = src.reshape(NW, H, NGC, GC)
    dst4 = dst.reshape(NW, H, NGC, GC)
    er5 = edge_reps.reshape(NW, H, JB, EB, D)

    w1t = W1.T
    ws3 = jnp.concatenate([Wk.T, Wv.T, w1t[:D]], axis=1)
    wd2 = jnp.concatenate([Wq.T, w1t[2 * D:]], axis=1)
    w1e = w1t[D:2 * D]
    b1r = b1.reshape(1, D)
    w2t = W2.T
    b2r = b2.reshape(1, D)

    z1 = jnp.zeros((N,), _f32)
    zn = jnp.zeros((N, D), _f32)

    trip = None
    dparts = []
    aggps = []
    for h in range(H):
        sr4, dr4 = _gather_rows(node_reps, src4[:, h], dst4[:, h])
        srb = sr4.reshape(NW, JB, EB, D)
        drb = dr4.reshape(NW, JB, EB, D)
        e4, ev4, trip = _edge_pass(h, srb, drb, er5, ws3, wd2, w1e,
                                   b1r, w2t, b2r, trip)
        d3 = dst4[:, h].reshape(NW, NSC, SCK)
        dparts.append(_denom(e4.reshape(NW, NSC, SCK), d3, z1))
        aggps.append(_agg_scatter(d3, ev4.reshape(EH, D), zn))

    dpa = dparts[0].reshape(NC, NNB, 1, NB)
    dpb = dparts[1].reshape(NC, NNB, 1, NB)
    updated = _final_pass(node_reps, aggps[0], aggps[1], dpa, dpb, Wo.T,
                          ln_scale.reshape(1, D), ln_bias.reshape(1, D))
    return (updated, trip.reshape(E, D))


# speedup vs baseline: 1.7614x; 1.7614x over previous
import functools
import math

import jax
import jax.numpy as jnp
from jax import lax
from jax.experimental import pallas as pl
from jax.experimental.pallas import tpu as pltpu
from jax.experimental.pallas import tpu_sc as plsc

N = 10000
E = 320000
D = 128

NC = 2
NS = 16
NW = NC * NS
EPW = E // NW
H = 2
EPWH = EPW // H
GC = 40
NGC = EPWH // GC
SCK = 40
NSC = EPWH // SCK
EH = E // H
NB = 2000
NNB = N // NB

_mesh = plsc.VectorSubcoreMesh(core_axis_name="c", subcore_axis_name="s")
_f32 = jnp.float32
_sc_params = pltpu.CompilerParams(needs_layout_passes=False)


_NBUF = 5


@functools.partial(
    pl.kernel,
    out_type=(
        jax.ShapeDtypeStruct((NW, EPWH, D), _f32),
        jax.ShapeDtypeStruct((NW, EPWH, D), _f32),
    ),
    mesh=_mesh,
    scratch_types=[
        pltpu.VMEM((NGC, GC), jnp.int32),
        [pltpu.VMEM((GC, D), _f32)] * _NBUF,
        [pltpu.SemaphoreType.DMA] * _NBUF,
        [pltpu.SemaphoreType.DMA] * _NBUF,
    ],
)
def _gather_rows(node_hbm, src3_hbm, dst3_hbm, sr_hbm, dr_hbm,
                 idx_v, bufs, gsems, wsems):
    cid = lax.axis_index("c")
    sid = lax.axis_index("s")
    wid = sid * NC + cid

    def run(idx3_hbm, out_hbm):
        pltpu.sync_copy(idx3_hbm.at[wid], idx_v)

        def gath(j, b):
            return pltpu.make_async_copy(
                node_hbm.at[idx_v.at[j]], bufs[b], gsems[b])

        def wrb(j, b):
            return pltpu.make_async_copy(
                bufs[b], out_hbm.at[wid, pl.ds(j * GC, GC)], wsems[b])

        for b in range(_NBUF):
            gath(b, b).start()

        def body(t, carry):
            j0 = _NBUF * t
            for b in range(_NBUF):
                gath(j0 + b, b).wait()
                wrb(j0 + b, b).start()
            for b in range(_NBUF):
                wrb(j0 + b, b).wait()

                @pl.when(j0 + b + _NBUF < NGC)
                def _():
                    gath(j0 + b + _NBUF, b).start()

            return carry

        lax.fori_loop(0, NGC // _NBUF, body, 0)

    run(src3_hbm, sr_hbm)
    run(dst3_hbm, dr_hbm)


def _edge_body(sr, dr, er, ws3, wd2, w1e, b1, w2t, b2, *rest):
    e_ref, ev_ref, t_ref = rest[-3:]
    s = sr[0]
    d = dr[0]
    ed = er[0, 0]
    s3 = jnp.dot(s, ws3[...], preferred_element_type=_f32)
    d2 = jnp.dot(d, wd2[...], preferred_element_type=_f32)
    k = s3[:, :D] + ed
    v = s3[:, D:2 * D] + ed
    q = d2[:, :D]
    ones_s = jnp.full((D, D), 1.0 / math.sqrt(D), _f32)
    e2d = jnp.exp(jnp.dot(q * k, ones_s, preferred_element_type=_f32))
    e_ref[0, 0, :] = e2d[:, 0]
    ev_ref[0] = v * e2d
    h = s3[:, 2 * D:] + jnp.dot(ed, w1e[...], preferred_element_type=_f32)
    h = jnp.maximum(h + d2[:, D:] + b1[...], 0.0)
    t_ref[0, 0] = jnp.dot(h, w2t[...], preferred_element_type=_f32) + b2[...]


def _edge_pass(h, srb, drb, er4, ws3, wd2, w1e, b1, w2t, b2, trip_in):
    eb3 = pl.BlockSpec((1, EPWH, D), lambda w: (w, 0, 0))
    eb4 = pl.BlockSpec((1, 1, EPWH, D), lambda w: (w, h, 0, 0))
    b_spec = pl.BlockSpec((1, D), lambda w: (0, 0))
    in_specs = [eb3, eb3, eb4,
                pl.BlockSpec((D, 3 * D), lambda w: (0, 0)),
                pl.BlockSpec((D, 2 * D), lambda w: (0, 0)),
                pl.BlockSpec((D, D), lambda w: (0, 0)),
                b_spec,
                pl.BlockSpec((D, D), lambda w: (0, 0)),
                b_spec]
    inputs = [srb, drb, er4, ws3, wd2, w1e, b1, w2t, b2]
    kwargs = {}
    if trip_in is not None:
        in_specs.append(pl.BlockSpec(memory_space=pl.ANY))
        inputs.append(trip_in)
        kwargs["input_output_aliases"] = {9: 2}
    return pl.pallas_call(
        _edge_body,
        grid=(NW,),
        in_specs=in_specs,
        out_specs=[
            pl.BlockSpec((1, 1, EPWH), lambda w: (w, 0, 0)),
            eb3,
            eb4,
        ],
        out_shape=[
            jax.ShapeDtypeStruct((NW, 1, EPWH), _f32),
            jax.ShapeDtypeStruct((NW, EPWH, D), _f32),
            jax.ShapeDtypeStruct((NW, H, EPWH, D), _f32),
        ],
        **kwargs,
    )(*inputs)


@functools.partial(
    pl.kernel,
    out_type=jax.ShapeDtypeStruct((NC, N), _f32),
    mesh=_mesh,
    scratch_types=[
        pltpu.VMEM((NSC, SCK), _f32),
        pltpu.VMEM((NSC, SCK), jnp.int32),
        pltpu.VMEM_SHARED((N,), _f32),
    ],
    compiler_params=_sc_params,
)
def _denom(e3_hbm, d3_hbm, z1_hbm, dpart_hbm, ebuf, dbuf, den_sh):
    cid = lax.axis_index("c")
    sid = lax.axis_index("s")
    wid = sid * NC + cid

    @pl.when(sid == 0)
    def _():
        pltpu.sync_copy(z1_hbm, den_sh)

    plsc.subcore_barrier()
    pltpu.sync_copy(e3_hbm.at[wid], ebuf)
    pltpu.sync_copy(d3_hbm.at[wid], dbuf)

    def body(j, carry):
        pltpu.sync_copy(ebuf.at[j], den_sh.at[dbuf.at[j]], add=True)
        return carry

    lax.fori_loop(0, NSC, body, 0)
    plsc.subcore_barrier()

    @pl.when(sid == 0)
    def _():
        pltpu.sync_copy(den_sh, dpart_hbm.at[cid])


@functools.partial(
    pl.kernel,
    out_type=jax.ShapeDtypeStruct((NC, N, D), _f32),
    mesh=_mesh,
    scratch_types=[
        pltpu.VMEM((NSC, SCK), jnp.int32),
        pltpu.VMEM((SCK, D), _f32),
        pltpu.VMEM((SCK, D), _f32),
        pltpu.VMEM_SHARED((N, D), _f32),
        pltpu.SemaphoreType.DMA,
        pltpu.SemaphoreType.DMA,
    ],
    compiler_params=_sc_params,
)
def _agg_scatter(d3_hbm, ev_hbm, zn_hbm, agg_hbm,
                 dbuf, buf_a, buf_b, agg_sh, sem_a, sem_b):
    cid = lax.axis_index("c")
    sid = lax.axis_index("s")
    wid = sid * NC + cid

    @pl.when(sid == 0)
    def _():
        pltpu.sync_copy(zn_hbm, agg_sh)

    pltpu.sync_copy(d3_hbm.at[wid], dbuf)
    plsc.subcore_barrier()

    def load(j, buf, sem):
        return pltpu.make_async_copy(
            ev_hbm.at[wid, pl.ds(j * SCK, SCK)], buf, sem)

    load(0, buf_a, sem_a).start()
    load(1, buf_b, sem_b).start()

    def body(t, carry):
        j0 = 2 * t
        load(j0, buf_a, sem_a).wait()
        pltpu.sync_copy(buf_a, agg_sh.at[dbuf.at[j0]], add=True)

        @pl.when(j0 + 2 < NSC)
        def _():
            load(j0 + 2, buf_a, sem_a).start()

        load(j0 + 1, buf_b, sem_b).wait()
        pltpu.sync_copy(buf_b, agg_sh.at[dbuf.at[j0 + 1]], add=True)

        @pl.when(j0 + 3 < NSC)
        def _():
            load(j0 + 3, buf_b, sem_b).start()

        return carry

    lax.fori_loop(0, NSC // 2, body, 0)
    load(NSC - 1, buf_a, sem_a).wait()
    pltpu.sync_copy(buf_a, agg_sh.at[dbuf.at[NSC - 1]], add=True)

    plsc.subcore_barrier()

    @pl.when(sid == 0)
    def _():
        pltpu.sync_copy(agg_sh, agg_hbm.at[cid])


def _final_body(node, aggpa, aggpb, dpa, dpb, wot, lns, lnb, out):
    den = (dpa[0, 0, 0, :] + dpa[1, 0, 0, :]
           + dpb[0, 0, 0, :] + dpb[1, 0, 0, :])
    rden = 1.0 / jnp.maximum(den, 1e-30)
    agg = (aggpa[0] + aggpa[1] + aggpb[0] + aggpb[1]) * rden[:, None]
    pre = node[...] + jnp.dot(agg, wot[...], preferred_element_type=_f32)
    mu = jnp.mean(pre, axis=1, keepdims=True)
    ctr = pre - mu
    var = jnp.mean(ctr * ctr, axis=1, keepdims=True)
    out[...] = ctr * lax.rsqrt(var + 1e-5) * lns[...] + lnb[...]


def _final_pass(node_reps, aggpa, aggpb, dpa, dpb, wot, lns, lnb):
    agg_spec = pl.BlockSpec((NC, NB, D), lambda i: (0, i, 0))
    dp_spec = pl.BlockSpec((NC, 1, 1, NB), lambda i: (0, i, 0, 0))
    return pl.pallas_call(
        _final_body,
        grid=(NNB,),
        in_specs=[
            pl.BlockSpec((NB, D), lambda i: (i, 0)),
            agg_spec, agg_spec, dp_spec, dp_spec,
            pl.BlockSpec((D, D), lambda i: (0, 0)),
            pl.BlockSpec((1, D), lambda i: (0, 0)),
            pl.BlockSpec((1, D), lambda i: (0, 0)),
        ],
        out_specs=pl.BlockSpec((NB, D), lambda i: (i, 0)),
        out_shape=jax.ShapeDtypeStruct((N, D), _f32),
    )(node_reps, aggpa, aggpb, dpa, dpb, wot, lns, lnb)


def kernel(node_reps, edge_reps, adjacency_list, Wq, Wk, Wv, Wo,
           ln_scale, ln_bias, W1, b1, W2, b2):
    src = adjacency_list[0]
    dst = adjacency_list[1]
    src4 = src.reshape(NW, H, NGC, GC)
    dst4 = dst.reshape(NW, H, NGC, GC)
    er4 = edge_reps.reshape(NW, H, EPWH, D)

    w1t = W1.T
    ws3 = jnp.concatenate([Wk.T, Wv.T, w1t[:D]], axis=1)
    wd2 = jnp.concatenate([Wq.T, w1t[2 * D:]], axis=1)
    w1e = w1t[D:2 * D]
    b1r = b1.reshape(1, D)
    w2t = W2.T
    b2r = b2.reshape(1, D)

    z1 = jnp.zeros((N,), _f32)
    zn = jnp.zeros((N, D), _f32)

    trip = None
    dparts = []
    aggps = []
    for h in range(H):
        srb, drb = _gather_rows(node_reps, src4[:, h], dst4[:, h])
        e3, ev3, trip = _edge_pass(h, srb, drb, er4, ws3, wd2, w1e,
                                   b1r, w2t, b2r, trip)
        d3 = dst4[:, h]
        dparts.append(_denom(e3.reshape(NW, NSC, SCK), d3, z1))
        aggps.append(_agg_scatter(d3, ev3, zn))

    dpa = dparts[0].reshape(NC, NNB, 1, NB)
    dpb = dparts[1].reshape(NC, NNB, 1, NB)
    updated = _final_pass(node_reps, aggps[0], aggps[1], dpa, dpb, Wo.T,
                          ln_scale.reshape(1, D), ln_bias.reshape(1, D))
    return (updated, trip.reshape(E, D))


# speedup vs baseline: 1.9780x; 1.1230x over previous
import functools
import math

import jax
import jax.numpy as jnp
from jax import lax
from jax.experimental import pallas as pl
from jax.experimental.pallas import tpu as pltpu
from jax.experimental.pallas import tpu_sc as plsc

N = 10000
E = 320000
D = 128

NC = 2
NS = 16
NW = NC * NS
EPW = E // NW
H = 2
EPWH = EPW // H
GC = 40
NGC = EPWH // GC
SCK = 40
NSC = EPWH // SCK
EH = E // H
NB = 2000
NNB = N // NB

_mesh = plsc.VectorSubcoreMesh(core_axis_name="c", subcore_axis_name="s")
_f32 = jnp.float32
_sc_params = pltpu.CompilerParams(needs_layout_passes=False)


_NBUF = 5


@functools.partial(
    pl.kernel,
    out_type=(
        jax.ShapeDtypeStruct((NW, EPWH, D), _f32),
        jax.ShapeDtypeStruct((NW, EPWH, D), _f32),
    ),
    mesh=_mesh,
    scratch_types=[
        pltpu.VMEM((NGC, GC), jnp.int32),
        [pltpu.VMEM((GC, D), _f32)] * _NBUF,
        [pltpu.SemaphoreType.DMA] * _NBUF,
        [pltpu.SemaphoreType.DMA] * _NBUF,
    ],
)
def _gather_rows(node_hbm, src3_hbm, dst3_hbm, sr_hbm, dr_hbm,
                 idx_v, bufs, gsems, wsems):
    cid = lax.axis_index("c")
    sid = lax.axis_index("s")
    wid = sid * NC + cid

    def run(idx3_hbm, out_hbm):
        pltpu.sync_copy(idx3_hbm.at[wid], idx_v)

        def gath(j, b):
            return pltpu.make_async_copy(
                node_hbm.at[idx_v.at[j]], bufs[b], gsems[b])

        def wrb(j, b):
            return pltpu.make_async_copy(
                bufs[b], out_hbm.at[wid, pl.ds(j * GC, GC)], wsems[b])

        for b in range(_NBUF):
            gath(b, b).start()

        def body(t, carry):
            j0 = _NBUF * t
            for b in range(_NBUF):
                gath(j0 + b, b).wait()
                wrb(j0 + b, b).start()
            for b in range(_NBUF):
                wrb(j0 + b, b).wait()

                @pl.when(j0 + b + _NBUF < NGC)
                def _():
                    gath(j0 + b + _NBUF, b).start()

            return carry

        lax.fori_loop(0, NGC // _NBUF, body, 0)

    run(src3_hbm, sr_hbm)
    run(dst3_hbm, dr_hbm)


def _edge_body(sr, dr, er, ws3, wd2, w1e, b1, w2t, b2, *rest):
    e_ref, ev_ref, t_ref = rest[-3:]
    s = sr[0]
    d = dr[0]
    ed = er[0, 0]
    s3 = jnp.dot(s, ws3[...], preferred_element_type=_f32)
    d2 = jnp.dot(d, wd2[...], preferred_element_type=_f32)
    k = s3[:, :D] + ed
    v = s3[:, D:2 * D] + ed
    q = d2[:, :D]
    ones_s = jnp.full((D, D), 1.0 / math.sqrt(D), _f32)
    e2d = jnp.exp(jnp.dot(q * k, ones_s, preferred_element_type=_f32))
    e_ref[0, 0, :] = e2d[:, 0]
    ev_ref[0] = v * e2d
    h = s3[:, 2 * D:] + jnp.dot(ed, w1e[...], preferred_element_type=_f32)
    h = jnp.maximum(h + d2[:, D:] + b1[...], 0.0)
    t_ref[0, 0] = jnp.dot(h, w2t[...], preferred_element_type=_f32) + b2[...]


def _edge_pass(h, srb, drb, er4, ws3, wd2, w1e, b1, w2t, b2, trip_in):
    eb3 = pl.BlockSpec((1, EPWH, D), lambda w: (w, 0, 0))
    eb4 = pl.BlockSpec((1, 1, EPWH, D), lambda w: (w, h, 0, 0))
    b_spec = pl.BlockSpec((1, D), lambda w: (0, 0))
    in_specs = [eb3, eb3, eb4,
                pl.BlockSpec((D, 3 * D), lambda w: (0, 0)),
                pl.BlockSpec((D, 2 * D), lambda w: (0, 0)),
                pl.BlockSpec((D, D), lambda w: (0, 0)),
                b_spec,
                pl.BlockSpec((D, D), lambda w: (0, 0)),
                b_spec]
    inputs = [srb, drb, er4, ws3, wd2, w1e, b1, w2t, b2]
    kwargs = {}
    if trip_in is not None:
        in_specs.append(pl.BlockSpec(memory_space=pl.ANY))
        inputs.append(trip_in)
        kwargs["input_output_aliases"] = {9: 2}
    return pl.pallas_call(
        _edge_body,
        grid=(NW,),
        in_specs=in_specs,
        out_specs=[
            pl.BlockSpec((1, 1, EPWH), lambda w: (w, 0, 0)),
            eb3,
            eb4,
        ],
        out_shape=[
            jax.ShapeDtypeStruct((NW, 1, EPWH), _f32),
            jax.ShapeDtypeStruct((NW, EPWH, D), _f32),
            jax.ShapeDtypeStruct((NW, H, EPWH, D), _f32),
        ],
        **kwargs,
    )(*inputs)


@functools.partial(
    pl.kernel,
    out_type=(
        jax.ShapeDtypeStruct((NC, N), _f32),
        jax.ShapeDtypeStruct((NC, N, D), _f32),
    ),
    mesh=_mesh,
    scratch_types=[
        pltpu.VMEM((NSC, SCK), jnp.int32),
        pltpu.VMEM((NSC, SCK), _f32),
        pltpu.VMEM((SCK, D), _f32),
        pltpu.VMEM((SCK, D), _f32),
        pltpu.VMEM_SHARED((N,), _f32),
        pltpu.VMEM_SHARED((N, D), _f32),
        pltpu.SemaphoreType.DMA,
        pltpu.SemaphoreType.DMA,
    ],
    compiler_params=_sc_params,
)
def _seg_scatter(d3_hbm, e3_hbm, ev_hbm, z1_hbm, zn_hbm, dpart_hbm, agg_hbm,
                 dbuf, ebuf, buf_a, buf_b, den_sh, agg_sh, sem_a, sem_b):
    cid = lax.axis_index("c")
    sid = lax.axis_index("s")
    wid = sid * NC + cid

    @pl.when(sid == 0)
    def _():
        pltpu.sync_copy(z1_hbm, den_sh)
        pltpu.sync_copy(zn_hbm, agg_sh)

    pltpu.sync_copy(d3_hbm.at[wid], dbuf)
    pltpu.sync_copy(e3_hbm.at[wid], ebuf)
    plsc.subcore_barrier()

    def load(j, buf, sem):
        return pltpu.make_async_copy(
            ev_hbm.at[wid, pl.ds(j * SCK, SCK)], buf, sem)

    load(0, buf_a, sem_a).start()
    load(1, buf_b, sem_b).start()

    def step(j, buf, sem):
        load(j, buf, sem).wait()
        pltpu.sync_copy(buf, agg_sh.at[dbuf.at[j]], add=True)
        pltpu.sync_copy(ebuf.at[j], den_sh.at[dbuf.at[j]], add=True)

        @pl.when(j + 2 < NSC)
        def _():
            load(j + 2, buf, sem).start()

    def body(t, carry):
        j0 = 2 * t
        step(j0, buf_a, sem_a)
        step(j0 + 1, buf_b, sem_b)
        return carry

    lax.fori_loop(0, NSC // 2, body, 0)
    load(NSC - 1, buf_a, sem_a).wait()
    pltpu.sync_copy(buf_a, agg_sh.at[dbuf.at[NSC - 1]], add=True)
    pltpu.sync_copy(ebuf.at[NSC - 1], den_sh.at[dbuf.at[NSC - 1]], add=True)

    plsc.subcore_barrier()

    @pl.when(sid == 0)
    def _():
        pltpu.sync_copy(den_sh, dpart_hbm.at[cid])
        pltpu.sync_copy(agg_sh, agg_hbm.at[cid])


def _final_body(node, aggpa, aggpb, dpa, dpb, wot, lns, lnb, out):
    den = (dpa[0, 0, 0, :] + dpa[1, 0, 0, :]
           + dpb[0, 0, 0, :] + dpb[1, 0, 0, :])
    rden = 1.0 / jnp.maximum(den, 1e-30)
    agg = (aggpa[0] + aggpa[1] + aggpb[0] + aggpb[1]) * rden[:, None]
    pre = node[...] + jnp.dot(agg, wot[...], preferred_element_type=_f32)
    mu = jnp.mean(pre, axis=1, keepdims=True)
    ctr = pre - mu
    var = jnp.mean(ctr * ctr, axis=1, keepdims=True)
    out[...] = ctr * lax.rsqrt(var + 1e-5) * lns[...] + lnb[...]


def _final_pass(node_reps, aggpa, aggpb, dpa, dpb, wot, lns, lnb):
    agg_spec = pl.BlockSpec((NC, NB, D), lambda i: (0, i, 0))
    dp_spec = pl.BlockSpec((NC, 1, 1, NB), lambda i: (0, i, 0, 0))
    return pl.pallas_call(
        _final_body,
        grid=(NNB,),
        in_specs=[
            pl.BlockSpec((NB, D), lambda i: (i, 0)),
            agg_spec, agg_spec, dp_spec, dp_spec,
            pl.BlockSpec((D, D), lambda i: (0, 0)),
            pl.BlockSpec((1, D), lambda i: (0, 0)),
            pl.BlockSpec((1, D), lambda i: (0, 0)),
        ],
        out_specs=pl.BlockSpec((NB, D), lambda i: (i, 0)),
        out_shape=jax.ShapeDtypeStruct((N, D), _f32),
    )(node_reps, aggpa, aggpb, dpa, dpb, wot, lns, lnb)


def kernel(node_reps, edge_reps, adjacency_list, Wq, Wk, Wv, Wo,
           ln_scale, ln_bias, W1, b1, W2, b2):
    src = adjacency_list[0]
    dst = adjacency_list[1]
    src4 = src.reshape(NW, H, NGC, GC)
    dst4 = dst.reshape(NW, H, NGC, GC)
    er4 = edge_reps.reshape(NW, H, EPWH, D)

    w1t = W1.T
    ws3 = jnp.concatenate([Wk.T, Wv.T, w1t[:D]], axis=1)
    wd2 = jnp.concatenate([Wq.T, w1t[2 * D:]], axis=1)
    w1e = w1t[D:2 * D]
    b1r = b1.reshape(1, D)
    w2t = W2.T
    b2r = b2.reshape(1, D)

    z1 = jnp.zeros((N,), _f32)
    zn = jnp.zeros((N, D), _f32)

    trip = None
    dparts = []
    aggps = []
    for h in range(H):
        srb, drb = _gather_rows(node_reps, src4[:, h], dst4[:, h])
        e3, ev3, trip = _edge_pass(h, srb, drb, er4, ws3, wd2, w1e,
                                   b1r, w2t, b2r, trip)
        d3 = dst4[:, h]
        dpart, aggp = _seg_scatter(d3, e3.reshape(NW, NSC, SCK), ev3, z1, zn)
        dparts.append(dpart)
        aggps.append(aggp)

    dpa = dparts[0].reshape(NC, NNB, 1, NB)
    dpb = dparts[1].reshape(NC, NNB, 1, NB)
    updated = _final_pass(node_reps, aggps[0], aggps[1], dpa, dpb, Wo.T,
                          ln_scale.reshape(1, D), ln_bias.reshape(1, D))
    return (updated, trip.reshape(E, D))
